# Initial kernel scaffold; baseline (speedup 1.0000x reference)
import jax
import jax.numpy as jnp
from jax.experimental import pallas as pl


def kernel(x, edge_index, edge_weight, W0_xz, W1_xz, b_xz, W0_hz, W1_hz, b_hz, W0_xr, W1_xr, b_xr, W0_hr, W1_hr, b_hr, W0_xh, W1_xh, b_xh, W0_hh, W1_hh, b_hh, W_lin, b_lin):
    raise NotImplementedError("write your pallas kernel here")


# speedup vs baseline: 56.2914x; 56.2914x over previous
import jax
import jax.numpy as jnp
from jax import lax
from jax.experimental import pallas as pl
from jax.experimental.pallas import tpu as pltpu
from jax.experimental.pallas import tpu_sc as plsc

NC = 2
NS = 16
NW = NC * NS
LANES = 16
LAGS = 8
F = 32
CHUNK = 128
SUP = 8


def _ceil_to(a, b):
    return (a + b - 1) // b * b


def _deg_sc(src2d, w2d, *, n_pad, rows_per_tile, nsup):
    npt = n_pad // NS

    def body(src_ref, w_ref, degp_ref, idx_v, w_v, zbuf, acc, sem):
        c = lax.axis_index("c")
        s = lax.axis_index("s")
        tid = c * NS + s

        def z16(i, _):
            zbuf[pl.ds(i * LANES, LANES)] = jnp.zeros((LANES,), jnp.float32)
            return 0
        lax.fori_loop(0, npt // LANES, z16, 0)
        pltpu.sync_copy(zbuf, acc.at[pl.ds(s * npt, npt)])
        plsc.subcore_barrier()

        def sup(k, _):
            rowbase = tid * rows_per_tile + k * SUP
            pltpu.sync_copy(src_ref.at[pl.ds(rowbase, SUP), :], idx_v)
            pltpu.sync_copy(w_ref.at[pl.ds(rowbase, SUP), :], w_v)
            descs = [
                pltpu.async_copy(w_v.at[j], acc.at[idx_v.at[j]], sem, add=True)
                for j in range(SUP)
            ]
            for d in descs:
                d.wait()
            return 0
        lax.fori_loop(0, nsup, sup, 0)
        plsc.subcore_barrier()
        pltpu.sync_copy(acc.at[pl.ds(s * npt, npt)],
                        degp_ref.at[c, pl.ds(s * npt, npt)])

    return pl.kernel(
        body,
        out_type=jax.ShapeDtypeStruct((NC, n_pad), jnp.float32),
        mesh=plsc.VectorSubcoreMesh(core_axis_name="c", subcore_axis_name="s"),
        scratch_types=[
            pltpu.VMEM((SUP, CHUNK), jnp.int32),
            pltpu.VMEM((SUP, CHUNK), jnp.float32),
            pltpu.VMEM((npt,), jnp.float32),
            pltpu.VMEM_SHARED((n_pad,), jnp.float32),
            pltpu.SemaphoreType.DMA,
        ],
        name="deg_scatter_sc",
    )(src2d, w2d)


def _msg_sc(src2d, dst2d, w2d, yT, *, n_pad, rows_per_tile, nsup):
    npt = n_pad // NS

    def body(src_ref, dst_ref, w_ref, y_ref, accp_ref,
             sidx, didx, wv, rbuf, zbuf, y0, y1, y2, y3, y4, y5, y6, y7,
             a0, a1, a2, a3, a4, a5, a6, a7, gsem, ssem):
        c = lax.axis_index("c")
        s = lax.axis_index("s")
        tid = c * NS + s
        ys = [y0, y1, y2, y3, y4, y5, y6, y7]
        accs = [a0, a1, a2, a3, a4, a5, a6, a7]
        sl = pl.ds(s * npt, npt)

        def z16(i, _):
            zbuf[pl.ds(i * LANES, LANES)] = jnp.zeros((LANES,), jnp.float32)
            return 0
        lax.fori_loop(0, npt // LANES, z16, 0)
        for f in range(LAGS):
            pltpu.sync_copy(y_ref.at[f, sl], ys[f].at[sl])
            pltpu.sync_copy(zbuf, accs[f].at[sl])
        plsc.subcore_barrier()

        def sup(k, _):
            rowbase = tid * rows_per_tile + k * SUP
            pltpu.sync_copy(src_ref.at[pl.ds(rowbase, SUP), :], sidx)
            pltpu.sync_copy(dst_ref.at[pl.ds(rowbase, SUP), :], didx)
            pltpu.sync_copy(w_ref.at[pl.ds(rowbase, SUP), :], wv)

            def chunk(j, _):
                gds = [
                    pltpu.async_copy(ys[f].at[sidx.at[j]], rbuf.at[f], gsem)
                    for f in range(LAGS)
                ]
                for d in gds:
                    d.wait()
                for q in range(CHUNK // LANES):
                    qs = pl.ds(q * LANES, LANES)
                    wq = wv[j, qs]
                    for f in range(LAGS):
                        rbuf[f, qs] = rbuf[f, qs] * wq
                sds = [
                    pltpu.async_copy(rbuf.at[f], accs[f].at[didx.at[j]],
                                     ssem, add=True)
                    for f in range(LAGS)
                ]
                for d in sds:
                    d.wait()
                return 0
            lax.fori_loop(0, SUP, chunk, 0)
            return 0
        lax.fori_loop(0, nsup, sup, 0)
        plsc.subcore_barrier()

        for f in range(LAGS):
            pltpu.sync_copy(accs[f].at[sl], accp_ref.at[c, f, sl])

    return pl.kernel(
        body,
        out_type=jax.ShapeDtypeStruct((NC, LAGS, n_pad), jnp.float32),
        mesh=plsc.VectorSubcoreMesh(core_axis_name="c", subcore_axis_name="s"),
        scratch_types=(
            [
                pltpu.VMEM((SUP, CHUNK), jnp.int32),
                pltpu.VMEM((SUP, CHUNK), jnp.int32),
                pltpu.VMEM((SUP, CHUNK), jnp.float32),
                pltpu.VMEM((LAGS, CHUNK), jnp.float32),
                pltpu.VMEM((npt,), jnp.float32),
            ]
            + [pltpu.VMEM_SHARED((n_pad,), jnp.float32)] * (2 * LAGS)
            + [pltpu.SemaphoreType.DMA, pltpu.SemaphoreType.DMA]
        ),
        name="msg_scatter_sc",
    )(src2d, dst2d, w2d, yT)


def _prep_tc(degp, xT, *, n_pad, blk):
    grid = n_pad // blk

    def body(degp_ref, xT_ref, dinvr_ref, yT_ref):
        deg = degp_ref[0:1, :] + degp_ref[1:2, :]
        safe = jnp.where(deg > 0, deg, 1.0)
        dinv = jnp.where(deg > 0, lax.rsqrt(safe), 0.0)
        dinvr_ref[...] = dinv
        yT_ref[...] = -dinv * xT_ref[...]

    return pl.pallas_call(
        body,
        grid=(grid,),
        in_specs=[
            pl.BlockSpec((NC, blk), lambda i: (0, i)),
            pl.BlockSpec((LAGS, blk), lambda i: (0, i)),
        ],
        out_specs=[
            pl.BlockSpec((1, blk), lambda i: (0, i)),
            pl.BlockSpec((LAGS, blk), lambda i: (0, i)),
        ],
        out_shape=[
            jax.ShapeDtypeStruct((1, n_pad), jnp.float32),
            jax.ShapeDtypeStruct((LAGS, n_pad), jnp.float32),
        ],
        name="prep_dinv_tc",
    )(degp, xT)


def _dense_tc(xT, dinvr, accp, wcatT, bcatT, wlinT, blin, *, n_pad, blk):
    grid = n_pad // blk

    def body(xT_ref, dinvr_ref, accp_ref, wcatT_ref, bcatT_ref, wlinT_ref,
             blin_ref, out_ref):
        stot = accp_ref[0] + accp_ref[1]
        tx1 = dinvr_ref[...] * stot
        a = jnp.concatenate([xT_ref[...], tx1], axis=0)
        g = jnp.dot(wcatT_ref[...], a,
                    preferred_element_type=jnp.float32) + bcatT_ref[...]
        z = jax.nn.sigmoid(g[:F, :])
        ht = jnp.tanh(g[F:, :])
        h = jax.nn.relu((1.0 - z) * ht)
        out_ref[...] = jnp.dot(wlinT_ref[...], h,
                               preferred_element_type=jnp.float32) + blin_ref[...]

    return pl.pallas_call(
        body,
        grid=(grid,),
        in_specs=[
            pl.BlockSpec((LAGS, blk), lambda i: (0, i)),
            pl.BlockSpec((1, blk), lambda i: (0, i)),
            pl.BlockSpec((NC, LAGS, blk), lambda i: (0, 0, i)),
            pl.BlockSpec((2 * F, 2 * LAGS), lambda i: (0, 0)),
            pl.BlockSpec((2 * F, 1), lambda i: (0, 0)),
            pl.BlockSpec((1, F), lambda i: (0, 0)),
            pl.BlockSpec((1, 1), lambda i: (0, 0)),
        ],
        out_specs=pl.BlockSpec((1, blk), lambda i: (0, i)),
        out_shape=jax.ShapeDtypeStruct((1, n_pad), jnp.float32),
        name="gates_dense_tc",
    )(xT, dinvr, accp, wcatT, bcatT, wlinT, blin)


def kernel(x, edge_index, edge_weight, W0_xz, W1_xz, b_xz, W0_hz, W1_hz, b_hz,
           W0_xr, W1_xr, b_xr, W0_hr, W1_hr, b_hr, W0_xh, W1_xh, b_xh,
           W0_hh, W1_hh, b_hh, W_lin, b_lin):
    n = x.shape[0]
    e = edge_weight.shape[0]
    n_pad = _ceil_to(n, NS * 8 * LANES)
    e_pad = _ceil_to(e, NW * SUP * CHUNK)
    rows_per_tile = e_pad // (NW * CHUNK)
    nsup = rows_per_tile // SUP

    padz = e_pad - e
    src = jnp.concatenate([edge_index[0], jnp.zeros((padz,), jnp.int32)])
    dst = jnp.concatenate([edge_index[1], jnp.zeros((padz,), jnp.int32)])
    w = jnp.concatenate([edge_weight, jnp.zeros((padz,), jnp.float32)])
    src2d = src.reshape(e_pad // CHUNK, CHUNK)
    dst2d = dst.reshape(e_pad // CHUNK, CHUNK)
    w2d = w.reshape(e_pad // CHUNK, CHUNK)
    xT = jnp.concatenate(
        [x, jnp.zeros((n_pad - n, LAGS), jnp.float32)]).T

    degp = _deg_sc(src2d, w2d, n_pad=n_pad, rows_per_tile=rows_per_tile,
                   nsup=nsup)

    dinvr, yT = _prep_tc(degp, xT, n_pad=n_pad, blk=2048)

    accp = _msg_sc(src2d, dst2d, w2d, yT, n_pad=n_pad,
                   rows_per_tile=rows_per_tile, nsup=nsup)

    wcatT = jnp.concatenate([
        jnp.concatenate([W0_xz, W0_xh], axis=1),
        jnp.concatenate([W1_xz, W1_xh], axis=1),
    ], axis=0).T
    bcatT = jnp.concatenate([b_xz + b_hz, b_xh + b_hh]).reshape(2 * F, 1)
    out2d = _dense_tc(xT, dinvr, accp, wcatT, bcatT, W_lin.T,
                      b_lin.reshape(1, 1), n_pad=n_pad, blk=2048)
    return out2d[0, :n].reshape(n, 1)
